# Initial kernel scaffold; baseline (speedup 1.0000x reference)
import jax
import jax.numpy as jnp
from jax.experimental import pallas as pl


def kernel(z, router_idx, W1, b1, W2, b2, centroids, tau_raw, gamma, beta):
    raise NotImplementedError("write your pallas kernel here")


# speedup vs baseline: 4.5199x; 4.5199x over previous
import functools
import math

import jax
import jax.numpy as jnp
from jax.experimental import pallas as pl
from jax.experimental.pallas import tpu as pltpu


def _fused_moe_body(ridx_ref, z_ref, c_ref, tau_ref, w1_ref, b1_ref,
                    w2_ref, b2_ref, gamma_ref, beta_ref,
                    out_ref, ew_ref):
    i = pl.program_id(0)
    E = c_ref.shape[1]
    B_total = pl.num_programs(0) * z_ref.shape[0]

    zb = z_ref[...]
    c = c_ref[0]

    t = tau_ref[ridx_ref[0], 0]
    tau = jnp.maximum(t, 0.0) + jnp.log1p(jnp.exp(-jnp.abs(t))) + 1e-6
    d2_cols = []
    for e in range(E):
        diff = zb - c[e:e + 1, :]
        d2_cols.append(jnp.sum(diff * diff, axis=1, keepdims=True))
    d2 = jnp.concatenate(d2_cols, axis=1)
    dist = jnp.sqrt(d2)
    scores = jnp.exp(-dist / tau)

    e_iota = jax.lax.broadcasted_iota(jnp.int32, scores.shape, 1)
    s0 = jnp.max(scores, axis=1, keepdims=True)
    i0 = jnp.argmax(scores, axis=1)[:, None]
    masked = jnp.where(e_iota == i0, -jnp.inf, scores)
    s1 = jnp.max(masked, axis=1, keepdims=True)
    i1 = jnp.argmax(masked, axis=1)[:, None]
    w0 = 1.0 / (1.0 + jnp.exp(s1 - s0))
    w1 = 1.0 - w0

    hits = ((i0 == e_iota).astype(jnp.float32) +
            (i1 == e_iota).astype(jnp.float32))
    cnt = jnp.sum(hits, axis=0, keepdims=True)

    @pl.when(i == 0)
    def _():
        ew_ref[...] = jnp.zeros_like(ew_ref)

    ew_ref[...] += cnt / (B_total * 2.0)

    zbf = zb.astype(jnp.bfloat16)
    acc = jnp.zeros(zb.shape, jnp.float32)
    inv_sqrt2 = 1.0 / math.sqrt(2.0)
    for e in range(E):
        h = jax.lax.dot_general(zbf, w1_ref[e], (((1,), (0,)), ((), ())),
                                preferred_element_type=jnp.float32)
        h = h + b1_ref[pl.ds(e, 1), :]
        h = h * 0.5 * (1.0 + jax.lax.erf(h * inv_sqrt2))
        o = jax.lax.dot_general(h.astype(jnp.bfloat16), w2_ref[e],
                                (((1,), (0,)), ((), ())),
                                preferred_element_type=jnp.float32)
        o = o + b2_ref[pl.ds(e, 1), :]
        coef = w0 * (i0 == e).astype(jnp.float32) + \
               w1 * (i1 == e).astype(jnp.float32)
        acc = acc + coef * o

    y = zb + acc
    mean = jnp.mean(y, axis=1, keepdims=True)
    yc = y - mean
    var = jnp.mean(yc * yc, axis=1, keepdims=True)
    out_ref[...] = yc * jax.lax.rsqrt(var + 1e-5) * gamma_ref[...] + beta_ref[...]


def kernel(z, router_idx, W1, b1, W2, b2, centroids, tau_raw, gamma, beta):
    B, D = z.shape
    E, _, H = W1.shape
    R = centroids.shape[0]
    TB = 512 if B % 512 == 0 else B
    NB = B // TB

    ridx = jnp.asarray(router_idx, jnp.int32).reshape((1,))
    w1_bf = W1.astype(jnp.bfloat16)
    w2_bf = W2.astype(jnp.bfloat16)
    tau2d = tau_raw.reshape(R, 1)

    grid_spec = pltpu.PrefetchScalarGridSpec(
        num_scalar_prefetch=1,
        grid=(NB,),
        in_specs=[
            pl.BlockSpec((TB, D), lambda i, r: (i, 0)),
            pl.BlockSpec((1, E, D), lambda i, r: (r[0], 0, 0)),
            pl.BlockSpec(memory_space=pltpu.SMEM),
            pl.BlockSpec((E, D, H), lambda i, r: (0, 0, 0)),
            pl.BlockSpec((E, H), lambda i, r: (0, 0)),
            pl.BlockSpec((E, H, D), lambda i, r: (0, 0, 0)),
            pl.BlockSpec((E, D), lambda i, r: (0, 0)),
            pl.BlockSpec((1, D), lambda i, r: (0, 0)),
            pl.BlockSpec((1, D), lambda i, r: (0, 0)),
        ],
        out_specs=[
            pl.BlockSpec((TB, D), lambda i, r: (i, 0)),
            pl.BlockSpec((1, E), lambda i, r: (0, 0)),
        ],
    )

    y_moe, ew = pl.pallas_call(
        _fused_moe_body,
        grid_spec=grid_spec,
        out_shape=[
            jax.ShapeDtypeStruct((B, D), jnp.float32),
            jax.ShapeDtypeStruct((1, E), jnp.float32),
        ],
    )(ridx, z, centroids, tau2d, w1_bf, b1, w2_bf, b2,
      gamma.reshape(1, D), beta.reshape(1, D))

    return y_moe, ew.reshape(E)
